# Initial kernel scaffold; baseline (speedup 1.0000x reference)
import jax
import jax.numpy as jnp
from jax.experimental import pallas as pl


def kernel(x, edge_index, W1, b1, W2, b2):
    raise NotImplementedError("write your pallas kernel here")


# speedup vs baseline: 17.7437x; 17.7437x over previous
import functools

import jax
import jax.numpy as jnp
from jax import lax
from jax.experimental import pallas as pl
from jax.experimental.pallas import tpu as pltpu
from jax.experimental.pallas import tpu_sc as plsc

N = 10000
NP = 10240
E = 320000
C = 128
NC = 2
NS = 16
NW = NC * NS
EPW = E // NW
BK = 80
NB = EPW // BK
RPS = NP // NS


def _sc_mesh():
    return plsc.VectorSubcoreMesh(core_axis_name="c", subcore_axis_name="s",
                                  num_cores=NC, num_subcores=NS)


@functools.cache
def _build_deg_kernel():
    @functools.partial(
        pl.kernel,
        out_type=jax.ShapeDtypeStruct((NC, NP, C), jnp.float32),
        mesh=_sc_mesh(),
        scratch_types=[
            pltpu.VMEM((NB, BK), jnp.int32),
            pltpu.VMEM((BK, C), jnp.float32),
            pltpu.VMEM_SHARED((NP, C), jnp.float32),
        ],
    )
    def _deg_kernel(dst_hbm, ones_hbm, zeros_hbm, out_hbm, didx, ones_v, acc_sh):
        c = lax.axis_index("c")
        s = lax.axis_index("s")
        wid = c * NS + s
        pltpu.sync_copy(zeros_hbm.at[pl.ds(s * RPS, RPS)],
                        acc_sh.at[pl.ds(s * RPS, RPS)])
        pltpu.sync_copy(ones_hbm, ones_v)
        pltpu.sync_copy(dst_hbm.at[wid], didx)
        plsc.subcore_barrier()

        def body(j, carry):
            pltpu.sync_copy(ones_v, acc_sh.at[didx.at[j]], add=True)
            return carry

        lax.fori_loop(0, NB, body, 0)
        plsc.subcore_barrier()
        pltpu.sync_copy(acc_sh.at[pl.ds(s * RPS, RPS)],
                        out_hbm.at[c, pl.ds(s * RPS, RPS)])

    return _deg_kernel


@functools.cache
def _build_scatter_kernel():
    @functools.partial(
        pl.kernel,
        out_type=jax.ShapeDtypeStruct((NC, NP, C), jnp.float32),
        mesh=_sc_mesh(),
        scratch_types=[
            pltpu.VMEM((NB, BK), jnp.int32),
            pltpu.VMEM((NB, BK), jnp.int32),
            pltpu.VMEM((BK, C), jnp.float32),
            pltpu.VMEM_SHARED((NP, C), jnp.float32),
            pltpu.SemaphoreType.DMA,
        ],
    )
    def _scatter_kernel(g_hbm, src_hbm, dst_hbm, zeros_hbm, out_hbm,
                        sidx, didx, rows, acc_sh, sem):
        c = lax.axis_index("c")
        s = lax.axis_index("s")
        wid = c * NS + s
        pltpu.sync_copy(zeros_hbm.at[pl.ds(s * RPS, RPS)],
                        acc_sh.at[pl.ds(s * RPS, RPS)])
        pltpu.sync_copy(src_hbm.at[wid], sidx)
        pltpu.sync_copy(dst_hbm.at[wid], didx)
        plsc.subcore_barrier()

        def body(j, carry):
            pltpu.async_copy(g_hbm.at[sidx.at[j]], rows, sem).wait()
            pltpu.sync_copy(rows, acc_sh.at[didx.at[j]], add=True)
            return carry

        lax.fori_loop(0, NB, body, 0)
        plsc.subcore_barrier()
        pltpu.sync_copy(acc_sh.at[pl.ds(s * RPS, RPS)],
                        out_hbm.at[c, pl.ds(s * RPS, RPS)])

    return _scatter_kernel


def _dinv(d_ref):
    deg = d_ref[:, 0:1] + d_ref[:, 1:2] + 1.0
    return lax.rsqrt(deg)


def _tc1_body(x_ref, w1_ref, d_ref, g1_ref):
    h = jnp.dot(x_ref[...], w1_ref[...], preferred_element_type=jnp.float32)
    g1_ref[...] = h * _dinv(d_ref)


def _tc2_body(sa_ref, sb_ref, g1_ref, d_ref, b1_ref, w2_ref, g2_ref):
    dinv = _dinv(d_ref)
    x2 = dinv * (sa_ref[...] + sb_ref[...] + g1_ref[...]) + b1_ref[...]
    x2 = jnp.maximum(x2, 0.0)
    h2 = jnp.dot(x2, w2_ref[...], preferred_element_type=jnp.float32)
    g2_ref[...] = h2 * dinv


def _tc3_body(sa_ref, sb_ref, g2_ref, d_ref, b2_ref, out_ref):
    dinv = _dinv(d_ref)
    out_ref[...] = dinv * (sa_ref[...] + sb_ref[...] + g2_ref[...]) + b2_ref[...]


_f32 = jnp.float32
_tc1 = pl.pallas_call(_tc1_body, out_shape=jax.ShapeDtypeStruct((N, C), _f32))
_tc2 = pl.pallas_call(_tc2_body, out_shape=jax.ShapeDtypeStruct((N, C), _f32))
_tc3 = pl.pallas_call(_tc3_body, out_shape=jax.ShapeDtypeStruct((N, C), _f32))


def kernel(x, edge_index, W1, b1, W2, b2):
    ei = edge_index.astype(jnp.int32)
    src3 = ei[0].reshape(NW, NB, BK)
    dst3 = ei[1].reshape(NW, NB, BK)
    onesC = jnp.ones((BK, C), _f32)
    zerosC = jnp.zeros((NP, C), _f32)
    b1r = b1.reshape(1, C)
    b2r = b2.reshape(1, C)

    deg_kernel = _build_deg_kernel()
    scatter_kernel = _build_scatter_kernel()

    degp = deg_kernel(dst3, onesC, zerosC)
    d2 = jnp.stack([degp[0, :N, 0], degp[1, :N, 0]], axis=1)

    g1 = _tc1(x, W1, d2)
    s1 = scatter_kernel(g1, src3, dst3, zerosC)
    g2 = _tc2(s1[0, :N], s1[1, :N], g1, d2, b1r, W2)
    s2 = scatter_kernel(g2, src3, dst3, zerosC)
    out = _tc3(s2[0, :N], s2[1, :N], g2, d2, b2r)
    return out
